# baseline (speedup 1.0000x reference)
import jax
import jax.numpy as jnp
from jax.experimental import pallas as pl
from jax.experimental.pallas import tpu as pltpu

_ALPHA = 0.5
_B, _H, _W, _D = 2, 48, 48, 48
_L = _W * _D
_N = _H * _W * _D
_BIG = jnp.iinfo(jnp.int32).max
_HC, _WC, _DC = _H // 2, _W // 2, _D // 2
_LC = _WC * _DC
_LP = 640
_DIRS = [(dh, dw, dd)
         for dh in (-1, 0, 1) for dw in (-1, 0, 1) for dd in (-1, 0, 1)
         if (dh, dw, dd) != (0, 0, 0)]


def _stencil(x, op, fill):
    shp = x.shape
    last = len(shp) - 1
    f = lambda s: jnp.full(s, fill, x.dtype)
    lo = jnp.concatenate([f(shp[:-2] + (1, shp[-1])), x[..., :-1, :]], axis=-2)
    hi = jnp.concatenate([x[..., 1:, :], f(shp[:-2] + (1, shp[-1]))], axis=-2)
    x = op(op(x, lo), hi)
    col = jax.lax.broadcasted_iota(jnp.int32, shp, last)
    lo = jnp.where(col < _D, fill, pltpu.roll(x, _D, last))
    hi = jnp.where(col >= _L - _D, fill, pltpu.roll(x, _L - _D, last))
    x = op(op(x, lo), hi)
    cm = col % _D
    lo = pltpu.roll(jnp.where(cm == _D - 1, fill, x), 1, last)
    hi = pltpu.roll(jnp.where(cm == 0, fill, x), _L - 1, last)
    x = op(op(x, lo), hi)
    return x


def _loss_kernel(p0_ref, p1_ref, t_ref, p0p_ref, p1p_ref, tp_ref, e1_ref,
                 f_ref, out_ref, labc_ref, a_ref, mv_ref, mm_ref):
    agp = jnp.logical_and(tp_ref[:] == 1, p1p_ref[:] > p0p_ref[:])
    c = {}
    for i in (0, 1):
        for j in (0, 1):
            for k in (0, 1):
                c[(i, j, k)] = agp[:, i * 4 + j * 2 + k]
    a_io = jax.lax.broadcasted_iota(jnp.int32, (_B, _HC, _LP), 1)
    colc = jax.lax.broadcasted_iota(jnp.int32, (_B, _HC, _LP), 2)
    b_io = colc // _DC
    c_io = colc % _DC
    lab0 = jnp.zeros((_B, _HC, _LP), jnp.int32)
    for (i, j, k), msk in c.items():
        idx = (2 * a_io + i) * _L + (2 * b_io + j) * _D + (2 * c_io + k) + 1
        lab0 = jnp.maximum(lab0, jnp.where(msk, idx, 0))
    labc_ref[:] = lab0

    S = dict(c)
    for i in (0, 1):
        for j in (0, 1):
            S[(i, j, 2)] = S[(i, j, 0)] | S[(i, j, 1)]
    for i in (0, 1):
        for k in (0, 1, 2):
            S[(i, 2, k)] = S[(i, 0, k)] | S[(i, 1, k)]
    for j in (0, 1, 2):
        for k in (0, 1, 2):
            S[(2, j, k)] = S[(0, j, k)] | S[(1, j, k)]

    def v_of(d):
        return tuple(1 if x > 0 else (0 if x < 0 else 2) for x in d)

    for di, (dh, dw, dd) in enumerate(_DIRS):
        src = S[v_of((-dh, -dw, -dd))].astype(jnp.int32)
        zr = jnp.zeros((_B, 1, _LP), jnp.int32)
        if dh == 1:
            src = jnp.concatenate([src[:, 1:, :], zr], axis=1)
        elif dh == -1:
            src = jnp.concatenate([zr, src[:, :-1, :]], axis=1)
        sh = (-(dw * _DC + dd)) % _LP
        if sh:
            src = pltpu.roll(src, sh, 2)
        ok = S[v_of((dh, dw, dd))] & (src != 0)
        if dh == 1:
            ok &= a_io < _HC - 1
        elif dh == -1:
            ok &= a_io > 0
        if dw == 1:
            ok &= b_io < _WC - 1
        elif dw == -1:
            ok &= b_io > 0
        if dd == 1:
            ok &= c_io < _DC - 1
        elif dd == -1:
            ok &= c_io > 0
        am = jnp.where(ok & (colc < _LC), jnp.int32(-1), jnp.int32(0))
        if sh:
            am = pltpu.roll(am, (_LP - sh) % _LP, 2)
        am = jnp.where(colc < _LC, am, 0)
        a_ref[di] = am

    didx = {d: i for i, d in enumerate(_DIRS)}

    zrow = jnp.zeros((_B, 1, _LP), jnp.int32)

    def step(lab):
        lab_u = jnp.concatenate([lab[:, 1:, :], zrow], axis=1)
        lab_d = jnp.concatenate([zrow, lab[:, :-1, :]], axis=1)
        hs = {1: lab_u, -1: lab_d, 0: lab}
        new = lab
        for dw in (-1, 0, 1):
            for dd in (-1, 0, 1):
                acc = None
                for dh in (-1, 0, 1):
                    if (dh, dw, dd) == (0, 0, 0):
                        continue
                    t = hs[dh] & a_ref[didx[(dh, dw, dd)]]
                    acc = t if acc is None else jnp.maximum(acc, t)
                sh = (-(dw * _DC + dd)) % _LP
                if sh:
                    acc = pltpu.roll(acc, sh, 2)
                new = jnp.maximum(new, acc)
        return new

    def ccl_body(_):
        lab = labc_ref[:]
        new = step(step(lab))
        labc_ref[:] = new
        return jnp.any(new != lab)

    jax.lax.while_loop(lambda go: go, ccl_body, jnp.bool_(True))

    labf = labc_ref[:].astype(jnp.float32).reshape(_B * _HC, _LP)
    dn = (((1,), (0,)), ((), ()))
    u = jax.lax.dot_general(f_ref[:], labf, dn,
                            preferred_element_type=jnp.float32)
    y = jax.lax.dot_general(u, e1_ref[:], dn,
                            preferred_element_type=jnp.float32)
    lab = y.astype(jnp.int32).reshape(_B, _H, _L)

    t1 = t_ref[:] == 1
    psq = p1_ref[:] > p0_ref[:]
    agree = jnp.logical_and(t1, psq)
    both = _stencil(jnp.concatenate(
        [jnp.where(agree, lab, 0), jnp.where(agree, _N + 1 - lab, 0)],
        axis=0), jnp.maximum, 0)
    nmax, nminc = both[0:_B], both[_B:2 * _B]

    fn = jnp.logical_and(t1, jnp.logical_not(psq))
    fp = jnp.logical_and(psq, jnp.logical_not(t1))

    mm_ref[:] = jnp.concatenate([fn, fn, fp, fp], axis=0).astype(jnp.int32)
    mv_ref[:] = jnp.concatenate(
        [jnp.where(fn, nminc, 0), jnp.where(fn, nmax, 0),
         jnp.where(fp, nminc, 0), jnp.where(fp, nmax, 0)], axis=0)

    def mist_body(_):
        v = mv_ref[:]
        mk = mm_ref[:] != 0
        new = jnp.where(mk, _stencil(v, jnp.maximum, 0), 0)
        mv_ref[:] = new
        return jnp.any(new != v)

    jax.lax.while_loop(lambda go: go, mist_body, jnp.bool_(True))

    v = mv_ref[:]
    cminc_fn, cmax_fn = v[0:_B], v[_B:2 * _B]
    cminc_fp, cmax_fp = v[2 * _B:3 * _B], v[3 * _B:4 * _B]
    single_fn = jnp.logical_and(cmax_fn > 0, (_N + 1 - cminc_fn) == cmax_fn)
    single_fp = jnp.logical_and(cmax_fp > 0, (_N + 1 - cminc_fp) == cmax_fp)
    crit_fn = jnp.logical_and(fn, jnp.logical_not(single_fn))
    crit_fp = jnp.logical_and(fp, jnp.logical_not(single_fp))
    combined = crit_fn.astype(jnp.float32) + crit_fp.astype(jnp.float32)

    p0, p1 = p0_ref[:], p1_ref[:]
    mx = jnp.maximum(p0, p1)
    lse = mx + jnp.log(jnp.exp(p0 - mx) + jnp.exp(p1 - mx))
    ce = lse - jnp.where(t1, p1, p0)
    total = jnp.sum((1.0 - _ALPHA + combined) * ce) / (_B * _N)
    out_ref[:] = total[None, None]


def _expand_mat():
    q = jnp.arange(_L, dtype=jnp.int32)[None, :]
    r = jnp.arange(_LP, dtype=jnp.int32)[:, None]
    return ((q // _D // 2) * _DC + (q % _D) // 2 == r).astype(jnp.float32)


def _hmat():
    q = jnp.arange(_B * _H, dtype=jnp.int32)[:, None]
    r = jnp.arange(_B * _HC, dtype=jnp.int32)[None, :]
    return ((q // _H) * _HC + (q % _H) // 2 == r).astype(jnp.float32)


def _parity(x):
    x = x.reshape(_B, _HC, 2, _WC, 2, _DC, 2)
    x = x.transpose(0, 2, 4, 6, 1, 3, 5)
    x = x.reshape(_B, 8, _HC, _LC)
    return jnp.pad(x, ((0, 0), (0, 0), (0, 0), (0, _LP - _LC)))


def kernel(preds, targets):
    p0 = preds[:, 0]
    p1 = preds[:, 1]
    t = targets[:, 0]
    out = pl.pallas_call(
        _loss_kernel,
        out_shape=jax.ShapeDtypeStruct((1, 1), jnp.float32),
        scratch_shapes=[
            pltpu.VMEM((_B, _HC, _LP), jnp.int32),
            pltpu.VMEM((len(_DIRS), _B, _HC, _LP), jnp.int32),
            pltpu.VMEM((4 * _B, _H, _L), jnp.int32),
            pltpu.VMEM((4 * _B, _H, _L), jnp.int32),
        ],
    )(p0.reshape(_B, _H, _L), p1.reshape(_B, _H, _L), t.reshape(_B, _H, _L),
      _parity(p0), _parity(p1), _parity(t), _expand_mat(), _hmat())
    return out[0, 0]


# speedup vs baseline: 4.5066x; 4.5066x over previous
import jax
import jax.numpy as jnp
from jax.experimental import pallas as pl
from jax.experimental.pallas import tpu as pltpu

_ALPHA = 0.5
_B, _H, _W, _D = 2, 48, 48, 48
_L = _W * _D
_N = _H * _W * _D
_M = (_H - 1) * _N + _N + 1


def _lane_masks(nd):
    col = jax.lax.broadcasted_iota(jnp.int32, (1,) * (nd - 1) + (_L,), nd - 1)
    cm = col % _D
    neg1 = jnp.int32(-1)
    mwlo = jnp.where(col >= _D, neg1, 0)
    mwhi = jnp.where(col < _L - _D, neg1, 0)
    mdlo = jnp.where(cm != _D - 1, neg1, 0)
    mdhi = jnp.where(cm != 0, neg1, 0)
    return mwlo, mwhi, mdlo, mdhi


def _stencil0(x, masks):
    mwlo, mwhi, mdlo, mdhi = masks
    shp = x.shape
    last = len(shp) - 1
    zrow = jnp.zeros(shp[:-2] + (1, shp[-1]), x.dtype)
    lo = jnp.concatenate([zrow, x[..., :-1, :]], axis=-2)
    hi = jnp.concatenate([x[..., 1:, :], zrow], axis=-2)
    x = jnp.maximum(jnp.maximum(x, lo), hi)
    lo = pltpu.roll(x, _D, last) & mwlo
    hi = pltpu.roll(x, _L - _D, last) & mwhi
    x = jnp.maximum(jnp.maximum(x, lo), hi)
    lo = pltpu.roll(x & mdlo, 1, last)
    hi = pltpu.roll(x & mdhi, _L - 1, last)
    x = jnp.maximum(jnp.maximum(x, lo), hi)
    return x


def _loss_kernel(p0_ref, p1_ref, t_ref, out_ref, lab_ref, m_ref, mv_ref, mm_ref):
    t1 = t_ref[:] == 1
    psq = p1_ref[:] > p0_ref[:]
    agree = jnp.logical_and(t1, psq)

    row = jax.lax.broadcasted_iota(jnp.int32, (_B, _H, _L), 1)
    col = jax.lax.broadcasted_iota(jnp.int32, (_B, _H, _L), 2)
    idx = row * _L + col + 1
    w_io = col // _D
    d_io = col % _D
    cheb = jnp.maximum(jnp.maximum(jnp.abs(2 * row - _H + 1),
                                   jnp.abs(2 * w_io - _W + 1)),
                       jnp.abs(2 * d_io - _D + 1))
    lab_ref[:] = jnp.where(agree, (_H - 1 - cheb) * _N + idx, 0)
    m_ref[:] = jnp.where(agree, jnp.int32(-1), 0)
    masks3 = _lane_masks(3)

    def ccl_body(_):
        lab = lab_ref[:]
        ag = m_ref[:]
        new = _stencil0(lab, masks3) & ag
        new = _stencil0(new, masks3) & ag
        lab_ref[:] = new
        return jnp.any(new != lab)

    jax.lax.while_loop(lambda go: go, ccl_body, jnp.bool_(True))

    lab = lab_ref[:]
    ag = m_ref[:]
    both = _stencil0(jnp.concatenate([lab, (_M - lab) & ag], axis=0), masks3)
    nmax, nminc = both[0:_B], both[_B:2 * _B]

    fn = jnp.logical_and(t1, jnp.logical_not(psq))
    fp = jnp.logical_and(psq, jnp.logical_not(t1))

    mm_ref[:] = jnp.where(jnp.concatenate([fn, fn, fp, fp], axis=0),
                          jnp.int32(-1), 0)
    mv_ref[:] = jnp.concatenate(
        [jnp.where(fn, nminc, 0), jnp.where(fn, nmax, 0),
         jnp.where(fp, nminc, 0), jnp.where(fp, nmax, 0)], axis=0)

    def mist_body(_):
        v = mv_ref[:]
        new = _stencil0(v, masks3) & mm_ref[:]
        mv_ref[:] = new
        return jnp.any(new != v)

    jax.lax.while_loop(lambda go: go, mist_body, jnp.bool_(True))

    v = mv_ref[:]
    cminc_fn, cmax_fn = v[0:_B], v[_B:2 * _B]
    cminc_fp, cmax_fp = v[2 * _B:3 * _B], v[3 * _B:4 * _B]
    single_fn = jnp.logical_and(cmax_fn > 0, (_M - cminc_fn) == cmax_fn)
    single_fp = jnp.logical_and(cmax_fp > 0, (_M - cminc_fp) == cmax_fp)
    crit_fn = jnp.logical_and(fn, jnp.logical_not(single_fn))
    crit_fp = jnp.logical_and(fp, jnp.logical_not(single_fp))
    combined = crit_fn.astype(jnp.float32) + crit_fp.astype(jnp.float32)

    p0, p1 = p0_ref[:], p1_ref[:]
    mx = jnp.maximum(p0, p1)
    lse = mx + jnp.log(jnp.exp(p0 - mx) + jnp.exp(p1 - mx))
    ce = lse - jnp.where(t1, p1, p0)
    total = jnp.sum((1.0 - _ALPHA + combined) * ce) / (_B * _N)
    out_ref[:] = total[None, None]


def kernel(preds, targets):
    p0 = preds[:, 0].reshape(_B, _H, _L)
    p1 = preds[:, 1].reshape(_B, _H, _L)
    t = targets[:, 0].reshape(_B, _H, _L)
    out = pl.pallas_call(
        _loss_kernel,
        out_shape=jax.ShapeDtypeStruct((1, 1), jnp.float32),
        scratch_shapes=[
            pltpu.VMEM((_B, _H, _L), jnp.int32),
            pltpu.VMEM((_B, _H, _L), jnp.int32),
            pltpu.VMEM((4 * _B, _H, _L), jnp.int32),
            pltpu.VMEM((4 * _B, _H, _L), jnp.int32),
        ],
    )(p0, p1, t)
    return out[0, 0]
